# baseline (speedup 1.0000x reference)
import functools

import jax
import jax.numpy as jnp
from jax.experimental import pallas as pl
from jax.experimental.pallas import tpu as pltpu


def _round_up(x, m):
    return ((x + m - 1) // m) * m


def _conv_kernel(x_ref, w_ref, o_ref, rhs_ref, *,
                 offsets, cin, m, kpad, hw, ho, w, wo, nb):
    kc = cin * len(offsets)
    rhs_ref[kc:kpad, :] = jnp.ones((kpad - kc, m), jnp.bfloat16)

    accs = []
    chunks = ((0, nb), (nb, m)) if nb < m else ((0, m),)
    for c0, c1 in chunks:
        for t, d in enumerate(offsets):
            lo = d + c0
            hi = min(d + c1, hw)
            rhs_ref[t * cin:(t + 1) * cin, c0:c0 + (hi - lo)] = x_ref[0, :, lo:hi]
        accs.append(jax.lax.dot_general(
            w_ref[...], rhs_ref[:, c0:c1],
            dimension_numbers=(((1,), (0,)), ((), ())),
            preferred_element_type=jnp.float32))

    if len(accs) == 1:
        accs = accs + accs
    for h in range(ho):
        a, b = h * w, h * w + wo
        if b <= nb:
            o_ref[0, :, h * wo:(h + 1) * wo] = accs[0][:, a:b]
        elif a >= nb:
            o_ref[0, :, h * wo:(h + 1) * wo] = accs[1][:, a - nb:b - nb]
        else:
            o_ref[0, :, h * wo:h * wo + (nb - a)] = accs[0][:, a:nb]
            o_ref[0, :, h * wo + (nb - a):(h + 1) * wo] = accs[1][:, :b - nb]


def kernel(x, weight, bias):
    N, Cin, H, W = x.shape
    Cout, Cin2, Kh, Kw = weight.shape
    assert Cin == Cin2
    Ho, Wo = H - Kh + 1, W - Kw + 1
    M = Ho * W
    offsets = tuple(kh * W + kw for kh in range(Kh) for kw in range(Kw))
    Kc = Cin * Kh * Kw
    Kpad = _round_up(Kc + 1, 8)
    NB = min(_round_up(M // 2, 128), M)

    w_mat = weight.transpose(0, 2, 3, 1).reshape(Cout, Kc)
    w_b = jnp.zeros((Cout, Kpad), jnp.bfloat16)
    w_b = w_b.at[:, :Kc].set(w_mat.astype(jnp.bfloat16))
    w_b = w_b.at[:, Kc].set(bias.astype(jnp.bfloat16))

    x_b = x.reshape(N, Cin, H * W).astype(jnp.bfloat16)

    out = pl.pallas_call(
        functools.partial(_conv_kernel, offsets=offsets, cin=Cin, m=M,
                          kpad=Kpad, hw=H * W, ho=Ho, w=W, wo=Wo, nb=NB),
        out_shape=jax.ShapeDtypeStruct((N, Cout, Ho * Wo), jnp.float32),
        grid=(N,),
        in_specs=[
            pl.BlockSpec((1, Cin, H * W), lambda n: (n, 0, 0)),
            pl.BlockSpec((Cout, Kpad), lambda n: (0, 0)),
        ],
        out_specs=pl.BlockSpec((1, Cout, Ho * Wo), lambda n: (n, 0, 0)),
        scratch_shapes=[
            pltpu.VMEM((Kpad, M), jnp.bfloat16),
        ],
        compiler_params=pltpu.CompilerParams(
            dimension_semantics=("parallel",),
        ),
    )(x_b, w_b)

    return out.reshape(N, Cout, Ho, Wo)


# speedup vs baseline: 1.0696x; 1.0696x over previous
import functools

import jax
import jax.numpy as jnp
from jax.experimental import pallas as pl
from jax.experimental.pallas import tpu as pltpu


def _round_up(x, m):
    return ((x + m - 1) // m) * m


def _conv_kernel(x_ref, w_ref, o_ref, xb_ref, rhs_a, rhs_b, *,
                 offsets, cin, m, kpad, hw, ho, w, wo, nb):
    kc = cin * len(offsets)
    xb_ref[...] = x_ref[0].astype(jnp.bfloat16)

    chunks = ((rhs_a, 0, nb), (rhs_b, nb, m)) if nb < m else ((rhs_a, 0, m),)
    accs = []
    for rhs_ref, c0, c1 in chunks:
        for t, d in enumerate(offsets):
            lo = d + c0
            hi = min(d + c1, hw)
            rhs_ref[t * cin:(t + 1) * cin, :hi - lo] = xb_ref[:, lo:hi]
        rhs_ref[kc:kpad, :] = jnp.ones((kpad - kc, c1 - c0), jnp.bfloat16)
        accs.append(jax.lax.dot_general(
            w_ref[...], rhs_ref[...],
            dimension_numbers=(((1,), (0,)), ((), ())),
            preferred_element_type=jnp.float32))

    if len(accs) == 1:
        accs = accs + accs
    for h in range(ho):
        a, b = h * w, h * w + wo
        if b <= nb:
            o_ref[0, :, h * wo:(h + 1) * wo] = accs[0][:, a:b]
        elif a >= nb:
            o_ref[0, :, h * wo:(h + 1) * wo] = accs[1][:, a - nb:b - nb]
        else:
            o_ref[0, :, h * wo:h * wo + (nb - a)] = accs[0][:, a:nb]
            o_ref[0, :, h * wo + (nb - a):(h + 1) * wo] = accs[1][:, :b - nb]


def kernel(x, weight, bias):
    N, Cin, H, W = x.shape
    Cout, Cin2, Kh, Kw = weight.shape
    assert Cin == Cin2
    Ho, Wo = H - Kh + 1, W - Kw + 1
    M = Ho * W
    offsets = tuple(kh * W + kw for kh in range(Kh) for kw in range(Kw))
    Kc = Cin * Kh * Kw
    Kpad = _round_up(Kc + 1, 8)
    NB = min(_round_up(M // 2, 128), M)

    w_mat = weight.transpose(0, 2, 3, 1).reshape(Cout, Kc)
    w_b = jnp.zeros((Cout, Kpad), jnp.bfloat16)
    w_b = w_b.at[:, :Kc].set(w_mat.astype(jnp.bfloat16))
    w_b = w_b.at[:, Kc].set(bias.astype(jnp.bfloat16))

    out = pl.pallas_call(
        functools.partial(_conv_kernel, offsets=offsets, cin=Cin, m=M,
                          kpad=Kpad, hw=H * W, ho=Ho, w=W, wo=Wo, nb=NB),
        out_shape=jax.ShapeDtypeStruct((N, Cout, Ho * Wo), jnp.float32),
        grid=(N,),
        in_specs=[
            pl.BlockSpec((1, Cin, H * W), lambda n: (n, 0, 0)),
            pl.BlockSpec((Cout, Kpad), lambda n: (0, 0)),
        ],
        out_specs=pl.BlockSpec((1, Cout, Ho * Wo), lambda n: (n, 0, 0)),
        scratch_shapes=[
            pltpu.VMEM((Cin, H * W), jnp.bfloat16),
            pltpu.VMEM((Kpad, NB), jnp.bfloat16),
            pltpu.VMEM((Kpad, max(M - NB, 8)), jnp.bfloat16),
        ],
        compiler_params=pltpu.CompilerParams(
            dimension_semantics=("parallel",),
        ),
    )(x.reshape(N, Cin, H * W), w_b)

    return out.reshape(N, Cout, Ho, Wo)


# speedup vs baseline: 1.0836x; 1.0131x over previous
import functools

import jax
import jax.numpy as jnp
from jax.experimental import pallas as pl
from jax.experimental.pallas import tpu as pltpu


def _round_up(x, m):
    return ((x + m - 1) // m) * m


def _conv_kernel(x_ref, w_ref, o_ref, xb_ref, rhs_ref, *,
                 offsets, cin, m, kpad, hw, ho, w, wo):
    kc = cin * len(offsets)
    xb_ref[...] = x_ref[0].astype(jnp.bfloat16)
    for t, d in enumerate(offsets):
        md = min(m, hw - d)
        rhs_ref[t * cin:(t + 1) * cin, :md] = xb_ref[:, d:d + md]
    rhs_ref[kc:kpad, :] = jnp.ones((kpad - kc, m), jnp.bfloat16)
    acc = jax.lax.dot_general(
        w_ref[...], rhs_ref[...],
        dimension_numbers=(((1,), (0,)), ((), ())),
        preferred_element_type=jnp.float32)
    for h in range(ho):
        o_ref[0, :, h * wo:(h + 1) * wo] = acc[:, h * w:h * w + wo]


def kernel(x, weight, bias):
    N, Cin, H, W = x.shape
    Cout, Cin2, Kh, Kw = weight.shape
    assert Cin == Cin2
    Ho, Wo = H - Kh + 1, W - Kw + 1
    M = Ho * W
    offsets = tuple(kh * W + kw for kh in range(Kh) for kw in range(Kw))
    Kc = Cin * Kh * Kw
    Kpad = _round_up(Kc + 1, 8)

    w_mat = weight.transpose(0, 2, 3, 1).reshape(Cout, Kc)
    w_b = jnp.zeros((Cout, Kpad), jnp.bfloat16)
    w_b = w_b.at[:, :Kc].set(w_mat.astype(jnp.bfloat16))
    w_b = w_b.at[:, Kc].set(bias.astype(jnp.bfloat16))

    out = pl.pallas_call(
        functools.partial(_conv_kernel, offsets=offsets, cin=Cin, m=M,
                          kpad=Kpad, hw=H * W, ho=Ho, w=W, wo=Wo),
        out_shape=jax.ShapeDtypeStruct((N, Cout, Ho * Wo), jnp.float32),
        grid=(N,),
        in_specs=[
            pl.BlockSpec((1, Cin, H * W), lambda n: (n, 0, 0)),
            pl.BlockSpec((Cout, Kpad), lambda n: (0, 0)),
        ],
        out_specs=pl.BlockSpec((1, Cout, Ho * Wo), lambda n: (n, 0, 0)),
        scratch_shapes=[
            pltpu.VMEM((Cin, H * W), jnp.bfloat16),
            pltpu.VMEM((Kpad, M), jnp.bfloat16),
        ],
        compiler_params=pltpu.CompilerParams(
            dimension_semantics=("parallel",),
        ),
    )(x.reshape(N, Cin, H * W), w_b)

    return out.reshape(N, Cout, Ho, Wo)


# speedup vs baseline: 1.1105x; 1.0248x over previous
import functools

import jax
import jax.numpy as jnp
from jax.experimental import pallas as pl
from jax.experimental.pallas import tpu as pltpu


def _round_up(x, m):
    return ((x + m - 1) // m) * m


def _conv_kernel(x_ref, w_ref, o_ref, xb_ref, rhs_ref, *,
                 offsets, cin, m, kpad, hw, ho, w, wo, bb):
    kc = cin * len(offsets)
    for b in range(bb):
        xb_ref[b] = x_ref[b].astype(jnp.bfloat16)
        for t, d in enumerate(offsets):
            md = min(m, hw - d)
            rhs_ref[t * cin:(t + 1) * cin, b * m:b * m + md] = (
                xb_ref[b, :, d:d + md])
    rhs_ref[kc:kpad, :] = jnp.ones((kpad - kc, bb * m), jnp.bfloat16)
    acc = jax.lax.dot_general(
        w_ref[...], rhs_ref[...],
        dimension_numbers=(((1,), (0,)), ((), ())),
        preferred_element_type=jnp.float32)
    for b in range(bb):
        for h in range(ho):
            o_ref[b, :, h * wo:(h + 1) * wo] = (
                acc[:, b * m + h * w:b * m + h * w + wo])


def kernel(x, weight, bias):
    N, Cin, H, W = x.shape
    Cout, Cin2, Kh, Kw = weight.shape
    assert Cin == Cin2
    Ho, Wo = H - Kh + 1, W - Kw + 1
    M = Ho * W
    offsets = tuple(kh * W + kw for kh in range(Kh) for kw in range(Kw))
    Kc = Cin * Kh * Kw
    Kpad = _round_up(Kc + 1, 8)
    B = 2 if N % 2 == 0 else 1

    w_mat = weight.transpose(0, 2, 3, 1).reshape(Cout, Kc)
    w_b = jnp.zeros((Cout, Kpad), jnp.bfloat16)
    w_b = w_b.at[:, :Kc].set(w_mat.astype(jnp.bfloat16))
    w_b = w_b.at[:, Kc].set(bias.astype(jnp.bfloat16))

    out = pl.pallas_call(
        functools.partial(_conv_kernel, offsets=offsets, cin=Cin, m=M,
                          kpad=Kpad, hw=H * W, ho=Ho, w=W, wo=Wo, bb=B),
        out_shape=jax.ShapeDtypeStruct((N, Cout, Ho * Wo), jnp.float32),
        grid=(N // B,),
        in_specs=[
            pl.BlockSpec((B, Cin, H * W), lambda n: (n, 0, 0)),
            pl.BlockSpec((Cout, Kpad), lambda n: (0, 0)),
        ],
        out_specs=pl.BlockSpec((B, Cout, Ho * Wo), lambda n: (n, 0, 0)),
        scratch_shapes=[
            pltpu.VMEM((B, Cin, H * W), jnp.bfloat16),
            pltpu.VMEM((Kpad, B * M), jnp.bfloat16),
        ],
        compiler_params=pltpu.CompilerParams(
            dimension_semantics=("parallel",),
        ),
    )(x.reshape(N, Cin, H * W), w_b)

    return out.reshape(N, Cout, Ho, Wo)


# speedup vs baseline: 1.1107x; 1.0002x over previous
import functools

import jax
import jax.numpy as jnp
from jax.experimental import pallas as pl
from jax.experimental.pallas import tpu as pltpu


def _round_up(x, m):
    return ((x + m - 1) // m) * m


def _conv_kernel(x_ref, w_ref, o_ref, xb_ref, rhs_ref, *,
                 offsets, cin, m, kpad, hw, ho, w, wo, bb):
    kc = cin * len(offsets)
    for b in range(bb):
        xb_ref[b] = x_ref[b].astype(jnp.bfloat16)
        for t, d in enumerate(offsets):
            md = min(m, hw - d)
            rhs_ref[t * cin:(t + 1) * cin, b * m:b * m + md] = (
                xb_ref[b, :, d:d + md])
    rhs_ref[kc:kpad, :] = jnp.ones((kpad - kc, bb * m), jnp.bfloat16)
    acc = jax.lax.dot_general(
        w_ref[...], rhs_ref[...],
        dimension_numbers=(((1,), (0,)), ((), ())),
        preferred_element_type=jnp.float32)
    for b in range(bb):
        for h in range(ho):
            o_ref[b, :, h * wo:(h + 1) * wo] = (
                acc[:, b * m + h * w:b * m + h * w + wo])


def kernel(x, weight, bias):
    N, Cin, H, W = x.shape
    Cout, Cin2, Kh, Kw = weight.shape
    assert Cin == Cin2
    Ho, Wo = H - Kh + 1, W - Kw + 1
    M = Ho * W
    offsets = tuple(kh * W + kw for kh in range(Kh) for kw in range(Kw))
    Kc = Cin * Kh * Kw
    Kpad = _round_up(Kc + 1, 8)
    B = 4 if N % 4 == 0 else (2 if N % 2 == 0 else 1)

    w_mat = weight.transpose(0, 2, 3, 1).reshape(Cout, Kc)
    w_b = jnp.zeros((Cout, Kpad), jnp.bfloat16)
    w_b = w_b.at[:, :Kc].set(w_mat.astype(jnp.bfloat16))
    w_b = w_b.at[:, Kc].set(bias.astype(jnp.bfloat16))

    out = pl.pallas_call(
        functools.partial(_conv_kernel, offsets=offsets, cin=Cin, m=M,
                          kpad=Kpad, hw=H * W, ho=Ho, w=W, wo=Wo, bb=B),
        out_shape=jax.ShapeDtypeStruct((N, Cout, Ho * Wo), jnp.float32),
        grid=(N // B,),
        in_specs=[
            pl.BlockSpec((B, Cin, H * W), lambda n: (n, 0, 0)),
            pl.BlockSpec((Cout, Kpad), lambda n: (0, 0)),
        ],
        out_specs=pl.BlockSpec((B, Cout, Ho * Wo), lambda n: (n, 0, 0)),
        scratch_shapes=[
            pltpu.VMEM((B, Cin, H * W), jnp.bfloat16),
            pltpu.VMEM((Kpad, B * M), jnp.bfloat16),
        ],
        compiler_params=pltpu.CompilerParams(
            dimension_semantics=("parallel",),
        ),
    )(x.reshape(N, Cin, H * W), w_b)

    return out.reshape(N, Cout, Ho, Wo)
